# baseline (speedup 1.0000x reference)
import jax
import jax.numpy as jnp
import numpy as np
from jax.experimental import pallas as pl

_N = 64
_D = 64
_F = 96
_ST = _N * _N
_BP = 32


def _body(xT_ref, wgf_ref, ee_ref,
          w1top_ref, w1bot_ref, b1cat_ref,
          w2T1_ref, b2col1_ref, w2T2_ref, b2col2_ref,
          dw1topT_ref, dw1botT_ref, db1col_ref, dw2T_ref, db2col_ref,
          out_ref):
    ee = ee_ref[...]
    trans_rhs = (((1,), (1,)), ((), ()))
    for j in range(_BP):
        x = xT_ref[j]

        a = jax.lax.dot_general(w1top_ref[...], x, trans_rhs,
                                preferred_element_type=jnp.float32)
        a = a + b1cat_ref[...]
        b = jax.lax.dot_general(w1bot_ref[...], x, trans_rhs,
                                preferred_element_type=jnp.float32)
        ab = jnp.concatenate([a, b], axis=1)
        h = jax.nn.relu(jnp.dot(ab, ee, preferred_element_type=jnp.float32))

        m1 = jnp.dot(w2T1_ref[...], h[:_F], preferred_element_type=jnp.float32)
        m1 = jax.nn.relu(m1 + b2col1_ref[...])
        m2 = jnp.dot(w2T2_ref[...], h[_F:], preferred_element_type=jnp.float32)
        m2 = jax.nn.relu(m2 + b2col2_ref[...])

        wgf = wgf_ref[j]
        w = m1 * wgf[0:1, :] + m2 * wgf[1:2, :]

        r = w[:, :2048] + w[:, 2048:]
        r = r[:, :1024] + r[:, 1024:]
        r = r[:, :512] + r[:, 512:]
        r = r[:, :256] + r[:, 256:]
        r = r[:, :128] + r[:, 128:]
        node_msgT = r[:, :_N] + r[:, _N:]

        d1T = (jax.lax.dot_general(dw1topT_ref[...], x, trans_rhs,
                                   preferred_element_type=jnp.float32)
               + jnp.dot(dw1botT_ref[...], node_msgT,
                         preferred_element_type=jnp.float32)
               + db1col_ref[...])
        d1T = jax.nn.relu(d1T)
        outT = jnp.dot(dw2T_ref[...], d1T, preferred_element_type=jnp.float32)
        outT = jax.nn.relu(outT + db2col_ref[...])
        out_ref[j] = outT.T


def kernel(node_states, edge_types, enc1_W1, enc1_b1, enc1_W2, enc1_b2,
           enc2_W1, enc2_b1, enc2_W2, enc2_b2, dec_W1, dec_b1, dec_W2, dec_b2):
    B = node_states.shape[0]
    x = node_states.reshape(B, _N, _D)

    et = edge_types[:, :, 1:, 0]
    et = jnp.transpose(et, (0, 2, 1))
    et = et.reshape(B, 2, _N - 1, _N)
    et = jnp.pad(et, ((0, 0), (0, 0), (0, 0), (0, 1)))
    et = et.reshape(B, 2, (_N - 1) * (_N + 1))
    wgf = jnp.pad(et, ((0, 0), (0, 0), (1, 0)))

    eye = np.eye(_N, dtype=np.float32)
    ee = jnp.asarray(np.concatenate([np.repeat(eye, _N, axis=1),
                                     np.tile(eye, (1, _N))], axis=0))

    w1top = jnp.concatenate([enc1_W1[:_D].T, enc2_W1[:_D].T], axis=0)
    w1bot = jnp.concatenate([enc1_W1[_D:].T, enc2_W1[_D:].T], axis=0)
    b1cat = jnp.concatenate([enc1_b1, enc2_b1]).reshape(2 * _F, 1)
    w2T1 = enc1_W2.T
    w2T2 = enc2_W2.T
    b2col1 = enc1_b2.reshape(_D, 1)
    b2col2 = enc2_b2.reshape(_D, 1)
    dw1topT = dec_W1[:_D].T
    dw1botT = dec_W1[_D:].T
    db1col = dec_b1.reshape(-1, 1)
    dw2T = dec_W2.T
    db2col = dec_b2.reshape(-1, 1)

    def full(a):
        return pl.BlockSpec(a.shape, lambda i: (0,) * a.ndim)

    consts = (ee, w1top, w1bot, b1cat, w2T1, b2col1, w2T2, b2col2,
              dw1topT, dw1botT, db1col, dw2T, db2col)

    out = pl.pallas_call(
        _body,
        grid=(B // _BP,),
        in_specs=[
            pl.BlockSpec((_BP, _N, _D), lambda i: (i, 0, 0)),
            pl.BlockSpec((_BP, 2, _ST), lambda i: (i, 0, 0)),
        ] + [full(w) for w in consts],
        out_specs=pl.BlockSpec((_BP, _N, _D), lambda i: (i, 0, 0)),
        out_shape=jax.ShapeDtypeStruct((B, _N, _D), jnp.float32),
    )(x, wgf, *consts)

    return out.reshape(B, _N, 1, _D)


# speedup vs baseline: 1.0944x; 1.0944x over previous
import jax
import jax.numpy as jnp
import numpy as np
from jax.experimental import pallas as pl

_N = 64
_D = 64
_F = 96
_ST = _N * _N
_BP = 8


def _body(xT_ref, wgf_ref, ee_ref,
          w1top_ref, w1bot_ref, b1cat_ref,
          w2T1_ref, b2col1_ref, w2T2_ref, b2col2_ref,
          dw1topT_ref, dw1botT_ref, db1col_ref, dw2T_ref, db2col_ref,
          out_ref):
    ee = ee_ref[...]
    trans_rhs = (((1,), (1,)), ((), ()))
    for j in range(_BP):
        x = xT_ref[j]

        a = jax.lax.dot_general(w1top_ref[...], x, trans_rhs,
                                preferred_element_type=jnp.float32)
        a = a + b1cat_ref[...]
        b = jax.lax.dot_general(w1bot_ref[...], x, trans_rhs,
                                preferred_element_type=jnp.float32)
        ab = jnp.concatenate([a, b], axis=1)
        h = jax.nn.relu(jnp.dot(ab, ee, preferred_element_type=jnp.float32))

        m1 = jnp.dot(w2T1_ref[...], h[:_F], preferred_element_type=jnp.float32)
        m1 = jax.nn.relu(m1 + b2col1_ref[...])
        m2 = jnp.dot(w2T2_ref[...], h[_F:], preferred_element_type=jnp.float32)
        m2 = jax.nn.relu(m2 + b2col2_ref[...])

        wgf = wgf_ref[j]
        w = m1 * wgf[0:1, :] + m2 * wgf[1:2, :]

        r = w[:, :2048] + w[:, 2048:]
        r = r[:, :1024] + r[:, 1024:]
        r = r[:, :512] + r[:, 512:]
        r = r[:, :256] + r[:, 256:]
        r = r[:, :128] + r[:, 128:]
        node_msgT = r[:, :_N] + r[:, _N:]

        d1T = (jax.lax.dot_general(dw1topT_ref[...], x, trans_rhs,
                                   preferred_element_type=jnp.float32)
               + jnp.dot(dw1botT_ref[...], node_msgT,
                         preferred_element_type=jnp.float32)
               + db1col_ref[...])
        d1T = jax.nn.relu(d1T)
        outT = jnp.dot(dw2T_ref[...], d1T, preferred_element_type=jnp.float32)
        outT = jax.nn.relu(outT + db2col_ref[...])
        out_ref[j] = outT.T


def kernel(node_states, edge_types, enc1_W1, enc1_b1, enc1_W2, enc1_b2,
           enc2_W1, enc2_b1, enc2_W2, enc2_b2, dec_W1, dec_b1, dec_W2, dec_b2):
    B = node_states.shape[0]
    x = node_states.reshape(B, _N, _D)

    et = edge_types[:, :, 1:, 0]
    et = jnp.transpose(et, (0, 2, 1))
    et = et.reshape(B, 2, _N - 1, _N)
    et = jnp.pad(et, ((0, 0), (0, 0), (0, 0), (0, 1)))
    et = et.reshape(B, 2, (_N - 1) * (_N + 1))
    wgf = jnp.pad(et, ((0, 0), (0, 0), (1, 0)))

    eye = np.eye(_N, dtype=np.float32)
    ee = jnp.asarray(np.concatenate([np.repeat(eye, _N, axis=1),
                                     np.tile(eye, (1, _N))], axis=0))

    w1top = jnp.concatenate([enc1_W1[:_D].T, enc2_W1[:_D].T], axis=0)
    w1bot = jnp.concatenate([enc1_W1[_D:].T, enc2_W1[_D:].T], axis=0)
    b1cat = jnp.concatenate([enc1_b1, enc2_b1]).reshape(2 * _F, 1)
    w2T1 = enc1_W2.T
    w2T2 = enc2_W2.T
    b2col1 = enc1_b2.reshape(_D, 1)
    b2col2 = enc2_b2.reshape(_D, 1)
    dw1topT = dec_W1[:_D].T
    dw1botT = dec_W1[_D:].T
    db1col = dec_b1.reshape(-1, 1)
    dw2T = dec_W2.T
    db2col = dec_b2.reshape(-1, 1)

    def full(a):
        return pl.BlockSpec(a.shape, lambda i: (0,) * a.ndim)

    consts = (ee, w1top, w1bot, b1cat, w2T1, b2col1, w2T2, b2col2,
              dw1topT, dw1botT, db1col, dw2T, db2col)

    out = pl.pallas_call(
        _body,
        grid=(B // _BP,),
        in_specs=[
            pl.BlockSpec((_BP, _N, _D), lambda i: (i, 0, 0)),
            pl.BlockSpec((_BP, 2, _ST), lambda i: (i, 0, 0)),
        ] + [full(w) for w in consts],
        out_specs=pl.BlockSpec((_BP, _N, _D), lambda i: (i, 0, 0)),
        out_shape=jax.ShapeDtypeStruct((B, _N, _D), jnp.float32),
    )(x, wgf, *consts)

    return out.reshape(B, _N, 1, _D)


# speedup vs baseline: 1.1030x; 1.0079x over previous
import jax
import jax.numpy as jnp
import numpy as np
from jax.experimental import pallas as pl

_N = 64
_D = 64
_F = 96
_ST = _N * _N
_BP = 16


def _body(xT_ref, wgf_ref, ee_ref,
          w1top_ref, w1bot_ref, b1cat_ref,
          w2T1_ref, b2col1_ref, w2T2_ref, b2col2_ref,
          dw1topT_ref, dw1botT_ref, db1col_ref, dw2T_ref, db2col_ref,
          out_ref):
    ee = ee_ref[...]
    trans_rhs = (((1,), (1,)), ((), ()))
    for j in range(_BP):
        x = xT_ref[j]

        a = jax.lax.dot_general(w1top_ref[...], x, trans_rhs,
                                preferred_element_type=jnp.float32)
        a = a + b1cat_ref[...]
        b = jax.lax.dot_general(w1bot_ref[...], x, trans_rhs,
                                preferred_element_type=jnp.float32)
        ab = jnp.concatenate([a, b], axis=1)
        h = jax.nn.relu(jnp.dot(ab, ee, preferred_element_type=jnp.float32))

        m1 = jnp.dot(w2T1_ref[...], h[:_F], preferred_element_type=jnp.float32)
        m1 = jax.nn.relu(m1 + b2col1_ref[...])
        m2 = jnp.dot(w2T2_ref[...], h[_F:], preferred_element_type=jnp.float32)
        m2 = jax.nn.relu(m2 + b2col2_ref[...])

        wgf = wgf_ref[j]
        w = m1 * wgf[0:1, :] + m2 * wgf[1:2, :]

        r = w[:, :2048] + w[:, 2048:]
        r = r[:, :1024] + r[:, 1024:]
        r = r[:, :512] + r[:, 512:]
        r = r[:, :256] + r[:, 256:]
        r = r[:, :128] + r[:, 128:]
        node_msgT = r[:, :_N] + r[:, _N:]

        d1T = (jax.lax.dot_general(dw1topT_ref[...], x, trans_rhs,
                                   preferred_element_type=jnp.float32)
               + jnp.dot(dw1botT_ref[...], node_msgT,
                         preferred_element_type=jnp.float32)
               + db1col_ref[...])
        d1T = jax.nn.relu(d1T)
        outT = jnp.dot(dw2T_ref[...], d1T, preferred_element_type=jnp.float32)
        outT = jax.nn.relu(outT + db2col_ref[...])
        out_ref[j] = outT.T


def kernel(node_states, edge_types, enc1_W1, enc1_b1, enc1_W2, enc1_b2,
           enc2_W1, enc2_b1, enc2_W2, enc2_b2, dec_W1, dec_b1, dec_W2, dec_b2):
    B = node_states.shape[0]
    x = node_states.reshape(B, _N, _D)

    et = edge_types[:, :, 1:, 0]
    et = jnp.transpose(et, (0, 2, 1))
    et = et.reshape(B, 2, _N - 1, _N)
    et = jnp.pad(et, ((0, 0), (0, 0), (0, 0), (0, 1)))
    et = et.reshape(B, 2, (_N - 1) * (_N + 1))
    wgf = jnp.pad(et, ((0, 0), (0, 0), (1, 0)))

    eye = np.eye(_N, dtype=np.float32)
    ee = jnp.asarray(np.concatenate([np.repeat(eye, _N, axis=1),
                                     np.tile(eye, (1, _N))], axis=0))

    w1top = jnp.concatenate([enc1_W1[:_D].T, enc2_W1[:_D].T], axis=0)
    w1bot = jnp.concatenate([enc1_W1[_D:].T, enc2_W1[_D:].T], axis=0)
    b1cat = jnp.concatenate([enc1_b1, enc2_b1]).reshape(2 * _F, 1)
    w2T1 = enc1_W2.T
    w2T2 = enc2_W2.T
    b2col1 = enc1_b2.reshape(_D, 1)
    b2col2 = enc2_b2.reshape(_D, 1)
    dw1topT = dec_W1[:_D].T
    dw1botT = dec_W1[_D:].T
    db1col = dec_b1.reshape(-1, 1)
    dw2T = dec_W2.T
    db2col = dec_b2.reshape(-1, 1)

    def full(a):
        return pl.BlockSpec(a.shape, lambda i: (0,) * a.ndim)

    consts = (ee, w1top, w1bot, b1cat, w2T1, b2col1, w2T2, b2col2,
              dw1topT, dw1botT, db1col, dw2T, db2col)

    out = pl.pallas_call(
        _body,
        grid=(B // _BP,),
        in_specs=[
            pl.BlockSpec((_BP, _N, _D), lambda i: (i, 0, 0)),
            pl.BlockSpec((_BP, 2, _ST), lambda i: (i, 0, 0)),
        ] + [full(w) for w in consts],
        out_specs=pl.BlockSpec((_BP, _N, _D), lambda i: (i, 0, 0)),
        out_shape=jax.ShapeDtypeStruct((B, _N, _D), jnp.float32),
    )(x, wgf, *consts)

    return out.reshape(B, _N, 1, _D)
